# baseline (device time: 710262 ns/iter reference)
import jax
import jax.numpy as jnp
from jax import lax
from jax.experimental import pallas as pl
from jax.experimental.pallas import tpu as pltpu

N_DEV = 32


def kernel(A, B):
    A16 = A.astype(jnp.bfloat16)
    B16 = B.astype(jnp.bfloat16)
    m, _ = A16.shape
    _, n = B16.shape
    chunk = m // N_DEV
    n_hops = N_DEV - 1

    def body(a_ref, b_ref, out_ref, comm_ref, agc_ref,
             rs_send, rs_recv, ag_send, ag_recv, rs_credit, ag_credit):
        d = lax.axis_index("i")
        left = (d - 1) % N_DEV
        right = (d + 1) % N_DEV

        def partial_chunk(c):
            a = a_ref[pl.ds(c * chunk, chunk), :]
            return jnp.dot(a, b_ref[:, :], preferred_element_type=jnp.float32)

        comm_ref[0, :, :] = partial_chunk(d)
        for h in range(n_hops):
            s, r = h % 2, (h + 1) % 2
            if h > 0:
                pl.semaphore_wait(rs_credit, 1)
            rdma = pltpu.make_async_remote_copy(
                src_ref=comm_ref.at[s],
                dst_ref=comm_ref.at[r],
                send_sem=rs_send.at[s],
                recv_sem=rs_recv.at[r],
                device_id=(right,),
                device_id_type=pl.DeviceIdType.MESH,
            )
            rdma.start()
            p = partial_chunk((d - h - 1) % N_DEV)
            rdma.wait()
            if h < n_hops - 1:
                pl.semaphore_signal(
                    rs_credit, inc=1,
                    device_id=(left,), device_id_type=pl.DeviceIdType.MESH,
                )
            comm_ref[r, :, :] = comm_ref[r, :, :] + p

        own = (d + 1) % N_DEV
        z = comm_ref[n_hops % 2, :, :]
        g = 0.5 * z * (1.0 + jnp.tanh(0.7978845608 * (z + 0.044715 * z * z * z)))
        g16 = g.astype(jnp.bfloat16)
        out_ref[pl.ds(own * chunk, chunk), :] = g16
        agc_ref[0, :, :] = g16

        for h in range(n_hops):
            s, r = h % 2, (h + 1) % 2
            if h > 0:
                pl.semaphore_wait(ag_credit, 1)
            rdma = pltpu.make_async_remote_copy(
                src_ref=agc_ref.at[s],
                dst_ref=agc_ref.at[r],
                send_sem=ag_send.at[s],
                recv_sem=ag_recv.at[r],
                device_id=(right,),
                device_id_type=pl.DeviceIdType.MESH,
            )
            rdma.start()
            rdma.wait()
            if h < n_hops - 1:
                pl.semaphore_signal(
                    ag_credit, inc=1,
                    device_id=(left,), device_id_type=pl.DeviceIdType.MESH,
                )
            rc = (d - h) % N_DEV
            out_ref[pl.ds(rc * chunk, chunk), :] = agc_ref[r, :, :]

    return pl.pallas_call(
        body,
        out_shape=jax.ShapeDtypeStruct((m, n), jnp.bfloat16),
        in_specs=[
            pl.BlockSpec(memory_space=pltpu.VMEM),
            pl.BlockSpec(memory_space=pltpu.VMEM),
        ],
        out_specs=pl.BlockSpec(memory_space=pltpu.VMEM),
        scratch_shapes=[
            pltpu.VMEM((2, chunk, n), jnp.float32),
            pltpu.VMEM((2, chunk, n), jnp.bfloat16),
            pltpu.SemaphoreType.DMA((2,)),
            pltpu.SemaphoreType.DMA((2,)),
            pltpu.SemaphoreType.DMA((2,)),
            pltpu.SemaphoreType.DMA((2,)),
            pltpu.SemaphoreType.REGULAR,
            pltpu.SemaphoreType.REGULAR,
        ],
    )(A16, B16)


# device time: 432308 ns/iter; 1.6430x vs baseline; 1.6430x over previous
import jax
import jax.numpy as jnp
from jax import lax
from jax.experimental import pallas as pl
from jax.experimental.pallas import tpu as pltpu

N_DEV = 32


CYC = [0, 3, 4, 7, 15, 12, 11, 8, 16, 19, 20, 23, 31, 28, 27, 24,
       25, 26, 29, 30, 22, 21, 18, 17, 9, 10, 13, 14, 6, 5, 2, 1]
POS = [0, 31, 30, 1, 2, 29, 28, 3, 7, 24, 25, 6, 5, 26, 27, 4,
       8, 23, 22, 9, 10, 21, 20, 11, 15, 16, 17, 14, 13, 18, 19, 12]


def kernel(A, B):
    A16 = A.astype(jnp.bfloat16)
    B16 = B.astype(jnp.bfloat16)
    m, _ = A16.shape
    _, n = B16.shape
    chunk = m // N_DEV
    n_hops = N_DEV - 1

    cyc_t = jnp.asarray(CYC, dtype=jnp.int32)
    pos_t = jnp.asarray(POS, dtype=jnp.int32)
    d = lax.axis_index("i")
    pos = pos_t[d]
    scalars = jnp.stack(
        [pos, cyc_t[(pos - 1) % N_DEV], cyc_t[(pos + 1) % N_DEV]]
    ).astype(jnp.int32)

    def body(a_ref, b_ref, sc_ref, out_ref, comm_ref, agc_ref,
             rs_send, rs_recv, ag_send, ag_recv, rs_credit, ag_credit):
        p = sc_ref[0]
        left = sc_ref[1]
        right = sc_ref[2]

        def partial_chunk(c):
            a = a_ref[pl.ds(c * chunk, chunk), :]
            return jnp.dot(a, b_ref[:, :], preferred_element_type=jnp.float32)

        comm_ref[0, :, :] = partial_chunk(p)
        for h in range(n_hops):
            s, r = h % 2, (h + 1) % 2
            if h > 0:
                pl.semaphore_wait(rs_credit, 1)
            rdma = pltpu.make_async_remote_copy(
                src_ref=comm_ref.at[s],
                dst_ref=comm_ref.at[r],
                send_sem=rs_send.at[s],
                recv_sem=rs_recv.at[r],
                device_id=(right,),
                device_id_type=pl.DeviceIdType.MESH,
            )
            rdma.start()
            pc = partial_chunk((p - h - 1) % N_DEV)
            rdma.wait()
            if h < n_hops - 1:
                pl.semaphore_signal(
                    rs_credit, inc=1,
                    device_id=(left,), device_id_type=pl.DeviceIdType.MESH,
                )
            comm_ref[r, :, :] = comm_ref[r, :, :] + pc

        own = (p + 1) % N_DEV
        z = comm_ref[n_hops % 2, :, :]
        g = 0.5 * z * (1.0 + jnp.tanh(0.7978845608 * (z + 0.044715 * z * z * z)))
        g16 = g.astype(jnp.bfloat16)
        out_ref[pl.ds(own * chunk, chunk), :] = g16
        agc_ref[0, :, :] = g16

        for h in range(n_hops):
            s, r = h % 2, (h + 1) % 2
            if h > 0:
                pl.semaphore_wait(ag_credit, 1)
            rdma = pltpu.make_async_remote_copy(
                src_ref=agc_ref.at[s],
                dst_ref=agc_ref.at[r],
                send_sem=ag_send.at[s],
                recv_sem=ag_recv.at[r],
                device_id=(right,),
                device_id_type=pl.DeviceIdType.MESH,
            )
            rdma.start()
            rdma.wait()
            if h < n_hops - 1:
                pl.semaphore_signal(
                    ag_credit, inc=1,
                    device_id=(left,), device_id_type=pl.DeviceIdType.MESH,
                )
            rc = (p - h) % N_DEV
            out_ref[pl.ds(rc * chunk, chunk), :] = agc_ref[r, :, :]

    return pl.pallas_call(
        body,
        out_shape=jax.ShapeDtypeStruct((m, n), jnp.bfloat16),
        in_specs=[
            pl.BlockSpec(memory_space=pltpu.VMEM),
            pl.BlockSpec(memory_space=pltpu.VMEM),
            pl.BlockSpec(memory_space=pltpu.SMEM),
        ],
        out_specs=pl.BlockSpec(memory_space=pltpu.VMEM),
        scratch_shapes=[
            pltpu.VMEM((2, chunk, n), jnp.float32),
            pltpu.VMEM((2, chunk, n), jnp.bfloat16),
            pltpu.SemaphoreType.DMA((2,)),
            pltpu.SemaphoreType.DMA((2,)),
            pltpu.SemaphoreType.DMA((2,)),
            pltpu.SemaphoreType.DMA((2,)),
            pltpu.SemaphoreType.REGULAR,
            pltpu.SemaphoreType.REGULAR,
        ],
    )(A16, B16, scalars)


# device time: 345079 ns/iter; 2.0583x vs baseline; 1.2528x over previous
import jax
import jax.numpy as jnp
from jax import lax
from jax.experimental import pallas as pl
from jax.experimental.pallas import tpu as pltpu

N_DEV = 32


CYC = [0, 3, 4, 7, 15, 12, 11, 8, 16, 19, 20, 23, 31, 28, 27, 24,
       25, 26, 29, 30, 22, 21, 18, 17, 9, 10, 13, 14, 6, 5, 2, 1]
POS = [0, 31, 30, 1, 2, 29, 28, 3, 7, 24, 25, 6, 5, 26, 27, 4,
       8, 23, 22, 9, 10, 21, 20, 11, 15, 16, 17, 14, 13, 18, 19, 12]


def kernel(A, B):
    A16 = A.astype(jnp.bfloat16)
    B16 = B.astype(jnp.bfloat16)
    m, _ = A16.shape
    _, n = B16.shape
    chunk = m // N_DEV
    n_hops = N_DEV - 1

    cyc_t = jnp.asarray(CYC, dtype=jnp.int32)
    pos_t = jnp.asarray(POS, dtype=jnp.int32)
    d = lax.axis_index("i")
    pos = pos_t[d]
    scalars = jnp.stack(
        [pos, cyc_t[(pos - 1) % N_DEV], cyc_t[(pos + 1) % N_DEV]]
    ).astype(jnp.int32)

    def body(a_ref, b_ref, sc_ref, out_ref, comm_ref, agc_ref,
             rs_send, rs_recv, ag_send, ag_recv, rs_credit, ag_credit):
        p = sc_ref[0]
        left = sc_ref[1]
        right = sc_ref[2]

        def partial_chunk(c):
            a = a_ref[pl.ds(c * chunk, chunk), :]
            return jnp.dot(a, b_ref[:, :], preferred_element_type=jnp.float32)

        comm_ref[0, :, :] = partial_chunk(p).astype(jnp.bfloat16)
        for h in range(n_hops - 1):
            s, r = h % 2, (h + 1) % 2
            if h > 0:
                pl.semaphore_wait(rs_credit, 1)
            rdma = pltpu.make_async_remote_copy(
                src_ref=comm_ref.at[s],
                dst_ref=comm_ref.at[r],
                send_sem=rs_send.at[s],
                recv_sem=rs_recv.at[r],
                device_id=(right,),
                device_id_type=pl.DeviceIdType.MESH,
            )
            rdma.start()
            pc = partial_chunk((p - h - 1) % N_DEV)
            rdma.wait()
            if h < n_hops - 1:
                pl.semaphore_signal(
                    rs_credit, inc=1,
                    device_id=(left,), device_id_type=pl.DeviceIdType.MESH,
                )
            comm_ref[r, :, :] = (comm_ref[r, :, :] + pc).astype(jnp.bfloat16)

        h = n_hops - 1
        s, r = h % 2, (h + 1) % 2
        pl.semaphore_wait(rs_credit, 1)
        rdma = pltpu.make_async_remote_copy(
            src_ref=comm_ref.at[s],
            dst_ref=comm_ref.at[r],
            send_sem=rs_send.at[s],
            recv_sem=rs_recv.at[r],
            device_id=(right,),
            device_id_type=pl.DeviceIdType.MESH,
        )
        rdma.start()
        pc = partial_chunk((p - h - 1) % N_DEV)
        rdma.wait()

        own = (p + 1) % N_DEV
        z = comm_ref[r, :, :] + pc
        g = 0.5 * z * (1.0 + jnp.tanh(0.7978845608 * (z + 0.044715 * z * z * z)))
        g16 = g.astype(jnp.bfloat16)
        out_ref[pl.ds(own * chunk, chunk), :] = g16
        agc_ref[0, :, :] = g16

        for h in range(n_hops):
            s, r = h % 2, (h + 1) % 2
            if h > 0:
                pl.semaphore_wait(ag_credit, 1)
            rdma = pltpu.make_async_remote_copy(
                src_ref=agc_ref.at[s],
                dst_ref=agc_ref.at[r],
                send_sem=ag_send.at[s],
                recv_sem=ag_recv.at[r],
                device_id=(right,),
                device_id_type=pl.DeviceIdType.MESH,
            )
            rdma.start()
            rdma.wait()
            if h < n_hops - 1:
                pl.semaphore_signal(
                    ag_credit, inc=1,
                    device_id=(left,), device_id_type=pl.DeviceIdType.MESH,
                )
            rc = (p - h) % N_DEV
            out_ref[pl.ds(rc * chunk, chunk), :] = agc_ref[r, :, :]

    return pl.pallas_call(
        body,
        out_shape=jax.ShapeDtypeStruct((m, n), jnp.bfloat16),
        in_specs=[
            pl.BlockSpec(memory_space=pltpu.VMEM),
            pl.BlockSpec(memory_space=pltpu.VMEM),
            pl.BlockSpec(memory_space=pltpu.SMEM),
        ],
        out_specs=pl.BlockSpec(memory_space=pltpu.VMEM),
        scratch_shapes=[
            pltpu.VMEM((2, chunk, n), jnp.bfloat16),
            pltpu.VMEM((2, chunk, n), jnp.bfloat16),
            pltpu.SemaphoreType.DMA((2,)),
            pltpu.SemaphoreType.DMA((2,)),
            pltpu.SemaphoreType.DMA((2,)),
            pltpu.SemaphoreType.DMA((2,)),
            pltpu.SemaphoreType.REGULAR,
            pltpu.SemaphoreType.REGULAR,
        ],
    )(A16, B16, scalars)
